# baseline (speedup 1.0000x reference)
import functools

import jax
import jax.numpy as jnp
from jax import lax
from jax.experimental import pallas as pl
from jax.experimental.pallas import tpu as pltpu
from jax.experimental.pallas import tpu_sc as plsc

B = 64
CH = 384
HW = 1024
ROWS = B * CH

_info = plsc.get_sparse_core_info()
NC = _info.num_cores
NS = _info.num_subcores
NW = NC * NS
RPW = ROWS // NW
K = 48
NCHUNK = RPW // K

_mesh = plsc.VectorSubcoreMesh(core_axis_name="c", subcore_axis_name="s")


@functools.partial(
    pl.kernel,
    mesh=_mesh,
    out_type=jax.ShapeDtypeStruct((ROWS, HW), jnp.float32),
    scratch_types=[
        pltpu.VMEM((NCHUNK, K), jnp.int32),
        pltpu.VMEM((K, HW), jnp.float32),
        pltpu.SemaphoreType.DMA,
    ],
)
def _sc_gather(val_hbm, rows_hbm, out_hbm, idx_v, buf, sem):
    wid = lax.axis_index("s") * NC + lax.axis_index("c")
    pltpu.sync_copy(rows_hbm.at[wid], idx_v)
    base = wid * RPW

    def step(i, carry):
        pltpu.async_copy(val_hbm.at[idx_v.at[i]], buf, sem).wait()
        pltpu.sync_copy(buf, out_hbm.at[pl.ds(base + i * K, K)])
        return carry

    lax.fori_loop(0, NCHUNK, step, 0, unroll=False)


def kernel(value, index):
    idx32 = index.astype(jnp.int32)
    rows = (jnp.arange(B, dtype=jnp.int32)[:, None] * CH + idx32[None, :]).reshape(
        NW, NCHUNK, K
    )
    val2 = value.reshape(ROWS, HW)
    out = _sc_gather(val2, rows)
    return out.reshape(B, CH, 32, 32)


# speedup vs baseline: 1.0058x; 1.0058x over previous
import functools

import jax
import jax.numpy as jnp
from jax import lax
from jax.experimental import pallas as pl
from jax.experimental.pallas import tpu as pltpu
from jax.experimental.pallas import tpu_sc as plsc

B = 64
CH = 384
HW = 1024
ROWS = B * CH

_info = plsc.get_sparse_core_info()
NC = _info.num_cores
NS = _info.num_subcores
NW = NC * NS
RPW = ROWS // NW
K = 32
NBUF = 3
NCHUNK = RPW // K
NG = NCHUNK // NBUF

_mesh = plsc.VectorSubcoreMesh(core_axis_name="c", subcore_axis_name="s")


@functools.partial(
    pl.kernel,
    mesh=_mesh,
    out_type=jax.ShapeDtypeStruct((ROWS, HW), jnp.float32),
    scratch_types=[
        pltpu.VMEM((NCHUNK, K), jnp.int32),
        pltpu.VMEM((K, HW), jnp.float32),
        pltpu.VMEM((K, HW), jnp.float32),
        pltpu.VMEM((K, HW), jnp.float32),
        pltpu.SemaphoreType.DMA,
        pltpu.SemaphoreType.DMA,
        pltpu.SemaphoreType.DMA,
        pltpu.SemaphoreType.DMA,
        pltpu.SemaphoreType.DMA,
        pltpu.SemaphoreType.DMA,
    ],
)
def _sc_gather(val_hbm, rows_hbm, out_hbm, idx_v, b0, b1, b2, g0, g1, g2, w0, w1, w2):
    bufs = (b0, b1, b2)
    gsems = (g0, g1, g2)
    wsems = (w0, w1, w2)
    wid = lax.axis_index("s") * NC + lax.axis_index("c")
    pltpu.sync_copy(rows_hbm.at[wid], idx_v)
    base = wid * RPW

    def gather(i, b):
        pltpu.async_copy(val_hbm.at[idx_v.at[i]], bufs[b], gsems[b])

    def writeback(i, b):
        pltpu.async_copy(bufs[b], out_hbm.at[pl.ds(base + i * K, K)], wsems[b])

    for b in range(NBUF):
        gather(b, b)

    def group(j, carry):
        i0 = j * NBUF
        for b in range(NBUF):
            pltpu.make_async_copy(val_hbm.at[idx_v.at[0]], bufs[b], gsems[b]).wait()
            writeback(i0 + b, b)
        for b in range(NBUF):
            pltpu.make_async_copy(
                bufs[b], out_hbm.at[pl.ds(base, K)], wsems[b]
            ).wait()
            gather(i0 + NBUF + b, b)
        return carry

    lax.fori_loop(0, NG - 1, group, 0, unroll=False)

    i0 = (NG - 1) * NBUF
    for b in range(NBUF):
        pltpu.make_async_copy(val_hbm.at[idx_v.at[0]], bufs[b], gsems[b]).wait()
        writeback(i0 + b, b)
    for b in range(NBUF):
        pltpu.make_async_copy(bufs[b], out_hbm.at[pl.ds(base, K)], wsems[b]).wait()


def kernel(value, index):
    idx32 = index.astype(jnp.int32)
    rows = (jnp.arange(B, dtype=jnp.int32)[:, None] * CH + idx32[None, :]).reshape(
        NW, NCHUNK, K
    )
    val2 = value.reshape(ROWS, HW)
    out = _sc_gather(val2, rows)
    return out.reshape(B, CH, 32, 32)


# speedup vs baseline: 1.7827x; 1.7725x over previous
import functools

import jax
import jax.numpy as jnp
from jax import lax
from jax.experimental import pallas as pl
from jax.experimental.pallas import tpu as pltpu
from jax.experimental.pallas import tpu_sc as plsc

B = 64
CH = 384
IMG = 32
PIX = B * IMG * IMG

_info = plsc.get_sparse_core_info()
NC = _info.num_cores
NS = _info.num_subcores
NW = NC * NS
PPW = PIX // NW
PX = 64
NB = PPW // PX
NCG = CH // 16

_mesh = plsc.VectorSubcoreMesh(core_axis_name="c", subcore_axis_name="s")


@functools.partial(
    pl.kernel,
    mesh=_mesh,
    compiler_params=pltpu.CompilerParams(needs_layout_passes=False),
    out_type=jax.ShapeDtypeStruct((PIX, CH), jnp.float32),
    scratch_types=[
        pltpu.VMEM((CH,), jnp.int32),
        pltpu.VMEM((PX, CH), jnp.float32),
        pltpu.VMEM((PX, CH), jnp.float32),
        pltpu.VMEM((PX, CH), jnp.float32),
        pltpu.VMEM((PX, CH), jnp.float32),
        pltpu.SemaphoreType.DMA,
        pltpu.SemaphoreType.DMA,
        pltpu.SemaphoreType.DMA,
        pltpu.SemaphoreType.DMA,
    ],
)
def _sc_permute(val_hbm, idx_hbm, out_hbm, idx_v, in0, in1, ou0, ou1, i0, i1, o0, o1):
    ins = (in0, in1)
    ous = (ou0, ou1)
    isems = (i0, i1)
    osems = (o0, o1)
    wid = lax.axis_index("s") * NC + lax.axis_index("c")
    base = wid * PPW
    pltpu.sync_copy(idx_hbm, idx_v)

    def start_in(i, b):
        pltpu.async_copy(val_hbm.at[pl.ds(base + i * PX, PX)], ins[b], isems[b])

    def wait_in(b):
        pltpu.make_async_copy(val_hbm.at[pl.ds(base, PX)], ins[b], isems[b]).wait()

    def start_out(i, b):
        pltpu.async_copy(ous[b], out_hbm.at[pl.ds(base + i * PX, PX)], osems[b])

    def wait_out(b):
        pltpu.make_async_copy(ous[b], out_hbm.at[pl.ds(base, PX)], osems[b]).wait()

    def compute(b):
        inb = ins[b]
        oub = ous[b]
        for ci in range(NCG):
            cvec = idx_v[pl.ds(ci * 16, 16)]

            def body(p, pvec, cvec=cvec, inb=inb, oub=oub, ci=ci):
                v = plsc.load_gather(inb, [pvec, cvec])
                oub[p, pl.ds(ci * 16, 16)] = v
                return pvec + 1

            lax.fori_loop(
                0, PX, body, jnp.zeros((16,), jnp.int32), unroll=4
            )

    start_in(0, 0)
    start_in(1, 1)

    def group(j, carry):
        ib = j * 2
        for b in range(2):
            wait_in(b)
            compute(b)
            start_out(ib + b, b)
        for b in range(2):
            wait_out(b)
            start_in(ib + 2 + b, b)
        return carry

    lax.fori_loop(0, NB // 2 - 1, group, 0, unroll=False)

    ib = NB - 2
    for b in range(2):
        wait_in(b)
        compute(b)
        start_out(ib + b, b)
    for b in range(2):
        wait_out(b)


def kernel(value, index):
    idx32 = index.astype(jnp.int32)
    pflat = value.transpose(0, 2, 3, 1).reshape(PIX, CH)
    out = _sc_permute(pflat, idx32)
    return out.reshape(B, IMG, IMG, CH).transpose(0, 3, 1, 2)


# speedup vs baseline: 1.8167x; 1.0190x over previous
import functools

import jax
import jax.numpy as jnp
from jax import lax
from jax.experimental import pallas as pl
from jax.experimental.pallas import tpu as pltpu
from jax.experimental.pallas import tpu_sc as plsc

B = 64
CH = 384
IMG = 32
PIX = B * IMG * IMG

_info = plsc.get_sparse_core_info()
NC = _info.num_cores
NS = _info.num_subcores
NW = NC * NS
PPW = PIX // NW
PX = 64
NB = PPW // PX
NCG = CH // 16

_mesh = plsc.VectorSubcoreMesh(core_axis_name="c", subcore_axis_name="s")


@functools.partial(
    pl.kernel,
    mesh=_mesh,
    compiler_params=pltpu.CompilerParams(needs_layout_passes=False),
    out_type=jax.ShapeDtypeStruct((PIX, CH), jnp.float32),
    scratch_types=[
        pltpu.VMEM((CH,), jnp.int32),
        pltpu.VMEM((PX, CH), jnp.float32),
        pltpu.VMEM((PX, CH), jnp.float32),
        pltpu.VMEM((PX, CH), jnp.float32),
        pltpu.VMEM((PX, CH), jnp.float32),
        pltpu.SemaphoreType.DMA,
        pltpu.SemaphoreType.DMA,
        pltpu.SemaphoreType.DMA,
        pltpu.SemaphoreType.DMA,
    ],
)
def _sc_permute(val_hbm, idx_hbm, out_hbm, idx_v, in0, in1, ou0, ou1, i0, i1, o0, o1):
    ins = (in0, in1)
    ous = (ou0, ou1)
    isems = (i0, i1)
    osems = (o0, o1)
    wid = lax.axis_index("s") * NC + lax.axis_index("c")
    base = wid * PPW
    pltpu.sync_copy(idx_hbm, idx_v)

    def start_in(i, b):
        pltpu.async_copy(val_hbm.at[pl.ds(base + i * PX, PX)], ins[b], isems[b])

    def wait_in(b):
        pltpu.make_async_copy(val_hbm.at[pl.ds(base, PX)], ins[b], isems[b]).wait()

    def start_out(i, b):
        pltpu.async_copy(ous[b], out_hbm.at[pl.ds(base + i * PX, PX)], osems[b])

    def wait_out(b):
        pltpu.make_async_copy(ous[b], out_hbm.at[pl.ds(base, PX)], osems[b]).wait()

    zeros16 = jnp.zeros((16,), jnp.int32)
    cvecs = [idx_v[pl.ds(ci * 16, 16)] for ci in range(NCG)]

    def compute(b):
        inb = ins[b]
        oub = ous[b]

        def body(p, pvec):
            for ci in range(NCG):
                v = plsc.load_gather(inb, [pvec, cvecs[ci]])
                oub[p, pl.ds(ci * 16, 16)] = v
            return pvec + 1

        lax.fori_loop(0, PX, body, zeros16, unroll=4)

    start_in(0, 0)
    start_in(1, 1)

    def group(j, carry):
        ib = j * 2
        for b in range(2):
            wait_in(b)
            compute(b)
            start_out(ib + b, b)
        for b in range(2):
            wait_out(b)
            start_in(ib + 2 + b, b)
        return carry

    lax.fori_loop(0, NB // 2 - 1, group, 0, unroll=False)

    ib = NB - 2
    for b in range(2):
        wait_in(b)
        compute(b)
        start_out(ib + b, b)
    for b in range(2):
        wait_out(b)


def kernel(value, index):
    idx32 = index.astype(jnp.int32)
    pflat = value.transpose(0, 2, 3, 1).reshape(PIX, CH)
    out = _sc_permute(pflat, idx32)
    return out.reshape(B, IMG, IMG, CH).transpose(0, 3, 1, 2)


# speedup vs baseline: 3.7009x; 2.0372x over previous
import functools

import jax
import jax.numpy as jnp
from jax import lax
from jax.experimental import pallas as pl
from jax.experimental.pallas import tpu as pltpu
from jax.experimental.pallas import tpu_sc as plsc

B = 64
CH = 384
IMG = 32
PIX = B * IMG * IMG

_info = plsc.get_sparse_core_info()
NC = _info.num_cores
NS = _info.num_subcores
NW = NC * NS
PPW = PIX // NW
PX = 64
NB = PPW // PX
NCG = CH // 16

_mesh = plsc.VectorSubcoreMesh(core_axis_name="c", subcore_axis_name="s")


@functools.partial(
    pl.kernel,
    mesh=_mesh,
    compiler_params=pltpu.CompilerParams(needs_layout_passes=False),
    out_type=jax.ShapeDtypeStruct((PIX, CH), jnp.float32),
    scratch_types=[
        pltpu.VMEM((CH,), jnp.int32),
        pltpu.VMEM((PX, CH), jnp.float32),
        pltpu.VMEM((PX, CH), jnp.float32),
        pltpu.VMEM((PX, CH), jnp.float32),
        pltpu.VMEM((PX, CH), jnp.float32),
        pltpu.SemaphoreType.DMA,
        pltpu.SemaphoreType.DMA,
        pltpu.SemaphoreType.DMA,
        pltpu.SemaphoreType.DMA,
    ],
)
def _sc_permute(val_hbm, idx_hbm, out_hbm, idx_v, in0, in1, ou0, ou1, i0, i1, o0, o1):
    ins = (in0, in1)
    ous = (ou0, ou1)
    isems = (i0, i1)
    osems = (o0, o1)
    wid = lax.axis_index("s") * NC + lax.axis_index("c")
    base = wid * PPW
    pltpu.sync_copy(idx_hbm, idx_v)

    def start_in(i, b):
        pltpu.async_copy(val_hbm.at[pl.ds(base + i * PX, PX)], ins[b], isems[b])

    def wait_in(b):
        pltpu.make_async_copy(val_hbm.at[pl.ds(base, PX)], ins[b], isems[b]).wait()

    def start_out(i, b):
        pltpu.async_copy(ous[b], out_hbm.at[pl.ds(base + i * PX, PX)], osems[b])

    def wait_out(b):
        pltpu.make_async_copy(ous[b], out_hbm.at[pl.ds(base, PX)], osems[b]).wait()

    zeros16 = jnp.zeros((16,), jnp.int32)
    cvecs = [idx_v[pl.ds(ci * 16, 16)] for ci in range(NCG)]

    def compute(b):
        inb = ins[b]
        oub = ous[b]

        def body(p, pvec):
            vals = [plsc.load_gather(inb, [pvec, cvecs[ci]]) for ci in range(NCG)]
            for ci in range(NCG):
                oub[p, pl.ds(ci * 16, 16)] = vals[ci]
            return pvec + 1

        lax.fori_loop(0, PX, body, zeros16, unroll=4)

    start_in(0, 0)
    start_in(1, 1)

    def group(j, carry):
        ib = j * 2
        for b in range(2):
            wait_in(b)
            compute(b)
            start_out(ib + b, b)
        for b in range(2):
            wait_out(b)
            start_in(ib + 2 + b, b)
        return carry

    lax.fori_loop(0, NB // 2 - 1, group, 0, unroll=False)

    ib = NB - 2
    for b in range(2):
        wait_in(b)
        compute(b)
        start_out(ib + b, b)
    for b in range(2):
        wait_out(b)


def kernel(value, index):
    idx32 = index.astype(jnp.int32)
    pflat = value.transpose(0, 2, 3, 1).reshape(PIX, CH)
    out = _sc_permute(pflat, idx32)
    return out.reshape(B, IMG, IMG, CH).transpose(0, 3, 1, 2)
